# baseline (speedup 1.0000x reference)
import functools

import jax
import jax.numpy as jnp
from jax.experimental import pallas as pl

_D_MODEL = 4096
_N_EXP = 64
_TOP_K = 8
_ID_BIAS = 1.0
_NEG_INF = float("-inf")


def _router_block(h_ref, w_ref, sel_ref, gate_ref):
    logits = jax.lax.dot_general(
        w_ref[...],
        h_ref[...],
        dimension_numbers=(((1,), (1,)), ((), ())),
        preferred_element_type=jnp.float32,
    )
    idx_col = jax.lax.broadcasted_iota(jnp.int32, (_N_EXP, 128), 0)[:, :1]
    logits = logits + jnp.where(idx_col == _N_EXP - 1, _ID_BIAS, 0.0)

    work = logits
    vals = []
    cnts = []
    for _ in range(_TOP_K):
        m = jnp.max(work, axis=0, keepdims=True)
        eq = work == m
        vals.append(m)
        cnts.append(jnp.sum(eq.astype(jnp.float32), axis=0, keepdims=True))
        work = jnp.where(eq, _NEG_INF, work)

    cum = cnts[0]
    thr = vals[0]
    gt_cnt = jnp.zeros_like(cum)
    for j in range(1, _TOP_K):
        below = cum < _TOP_K
        thr = jnp.where(below, vals[j], thr)
        gt_cnt = jnp.where(below, cum, gt_cnt)
        cum = cum + cnts[j]

    eq_thr = logits == thr
    row_i = jax.lax.broadcasted_iota(jnp.int32, (_N_EXP, _N_EXP), 0)
    col_i = jax.lax.broadcasted_iota(jnp.int32, (_N_EXP, _N_EXP), 1)
    ltri = (row_i >= col_i).astype(jnp.float32)
    prefix = jax.lax.dot_general(
        ltri,
        eq_thr.astype(jnp.float32),
        dimension_numbers=(((1,), (0,)), ((), ())),
        preferred_element_type=jnp.float32,
    )
    need = _TOP_K - gt_cnt
    sel = jnp.where(
        (logits > thr) | (eq_thr & (prefix <= need)), 1.0, 0.0
    )

    sel_t = sel.T
    sel_ref[...] = sel_t
    gate_ref[...] = sel_t * (1.0 / _TOP_K)


@functools.partial(jax.jit, static_argnames=("block_rows",))
def _router(h, W, block_rows=1024):
    n_rows = h.shape[0]
    grid = (n_rows // block_rows,)
    sel, gate = pl.pallas_call(
        _router_block,
        grid=grid,
        in_specs=[
            pl.BlockSpec((block_rows, _D_MODEL), lambda i: (i, 0)),
            pl.BlockSpec((_N_EXP, _D_MODEL), lambda i: (0, 0)),
        ],
        out_specs=[
            pl.BlockSpec((block_rows, _N_EXP), lambda i: (i, 0)),
            pl.BlockSpec((block_rows, _N_EXP), lambda i: (i, 0)),
        ],
        out_shape=[
            jax.ShapeDtypeStruct((n_rows, _N_EXP), jnp.float32),
            jax.ShapeDtypeStruct((n_rows, _N_EXP), jnp.float32),
        ],
    )(h, W)
    return sel, gate


def kernel(h, W):
    sel, gate = _router(h, W)
    return sel.astype(bool), gate


# speedup vs baseline: 1.0009x; 1.0009x over previous
import functools

import jax
import jax.numpy as jnp
from jax.experimental import pallas as pl

_D_MODEL = 4096
_N_EXP = 64
_TOP_K = 8
_ID_BIAS = 1.0
_NEG_INF = float("-inf")


def _router_block(h_ref, w_ref, sel_ref, gate_ref):
    logits = jax.lax.dot_general(
        w_ref[...],
        h_ref[...],
        dimension_numbers=(((1,), (1,)), ((), ())),
        preferred_element_type=jnp.float32,
    )
    idx_col = jax.lax.broadcasted_iota(jnp.int32, (_N_EXP, 128), 0)[:, :1]
    logits = logits + jnp.where(idx_col == _N_EXP - 1, _ID_BIAS, 0.0)

    work = logits
    vals = []
    cnts = []
    for _ in range(_TOP_K):
        m = jnp.max(work, axis=0, keepdims=True)
        eq = work == m
        vals.append(m)
        cnts.append(jnp.sum(eq.astype(jnp.float32), axis=0, keepdims=True))
        work = jnp.where(eq, _NEG_INF, work)

    cum = cnts[0]
    thr = vals[0]
    gt_cnt = jnp.zeros_like(cum)
    for j in range(1, _TOP_K):
        below = cum < _TOP_K
        thr = jnp.where(below, vals[j], thr)
        gt_cnt = jnp.where(below, cum, gt_cnt)
        cum = cum + cnts[j]

    eq_thr = logits == thr
    row_i = jax.lax.broadcasted_iota(jnp.int32, (_N_EXP, _N_EXP), 0)
    col_i = jax.lax.broadcasted_iota(jnp.int32, (_N_EXP, _N_EXP), 1)
    ltri = (row_i >= col_i).astype(jnp.float32)
    prefix = jax.lax.dot_general(
        ltri,
        eq_thr.astype(jnp.float32),
        dimension_numbers=(((1,), (0,)), ((), ())),
        preferred_element_type=jnp.float32,
    )
    need = _TOP_K - gt_cnt
    sel = jnp.where(
        (logits > thr) | (eq_thr & (prefix <= need)), 1.0, 0.0
    )

    sel_t = sel.T
    sel_ref[...] = sel_t != 0.0
    gate_ref[...] = sel_t * (1.0 / _TOP_K)


@functools.partial(jax.jit, static_argnames=("block_rows",))
def _router(h, W, block_rows=1024):
    n_rows = h.shape[0]
    grid = (n_rows // block_rows,)
    sel, gate = pl.pallas_call(
        _router_block,
        grid=grid,
        in_specs=[
            pl.BlockSpec((block_rows, _D_MODEL), lambda i: (i, 0)),
            pl.BlockSpec((_N_EXP, _D_MODEL), lambda i: (0, 0)),
        ],
        out_specs=[
            pl.BlockSpec((block_rows, _N_EXP), lambda i: (i, 0)),
            pl.BlockSpec((block_rows, _N_EXP), lambda i: (i, 0)),
        ],
        out_shape=[
            jax.ShapeDtypeStruct((n_rows, _N_EXP), jnp.bool_),
            jax.ShapeDtypeStruct((n_rows, _N_EXP), jnp.float32),
        ],
    )(h, W)
    return sel, gate


def kernel(h, W):
    mask, gate = _router(h, W)
    return mask, gate
